# baseline (speedup 1.0000x reference)
import functools

import jax
import jax.numpy as jnp
from jax import lax
from jax.experimental import pallas as pl
from jax.experimental.pallas import tpu as pltpu
from jax.experimental.pallas import tpu_sc as plsc

N = 10000
E = 320000
FIN = 512
FM = 256
H = 128
G = 64
NCLS = 10

NSUB = 16
NCORE = 2
NW = NSUB * NCORE

CHUNK = 112
NPH = 5
NCH = 90
CPP = NCH // NPH
NBUF = 3
E_PAD = NW * NCH * CHUNK
N_PAD = 10112
RPT = N_PAD // NSUB


def _seg_sum_sc():
    mesh = plsc.VectorSubcoreMesh(core_axis_name="c", subcore_axis_name="s")

    @functools.partial(
        pl.kernel,
        out_type=(
            jax.ShapeDtypeStruct((N_PAD, H), jnp.float32),
            jax.ShapeDtypeStruct((N_PAD, H), jnp.float32),
        ),
        mesh=mesh,
        scratch_types=[
            pltpu.VMEM_SHARED((N_PAD, H), jnp.float32),
            pltpu.VMEM((CPP, CHUNK), jnp.int32),
            pltpu.VMEM((CPP, CHUNK), jnp.int32),
            pltpu.VMEM((NBUF, CHUNK, H), jnp.float32),
            [pltpu.SemaphoreType.DMA] * NBUF,
        ],
    )
    def seg_sum(h_hbm, srcm_hbm, dstm_hbm, zeros_hbm, out0, out1,
                agg, src_idx, dst_idx, rows, gsems):
        c = lax.axis_index("c")
        s = lax.axis_index("s")
        wid = c * NSUB + s
        pltpu.sync_copy(zeros_hbm.at[pl.ds(s * RPT, RPT)],
                        agg.at[pl.ds(s * RPT, RPT)])
        plsc.subcore_barrier()

        def gstart(j, b):
            pltpu.make_async_copy(
                h_hbm.at[src_idx.at[j]], rows.at[b], gsems[b]).start()

        def gwait(j, b):
            pltpu.make_async_copy(
                h_hbm.at[src_idx.at[j]], rows.at[b], gsems[b]).wait()

        for ph in range(NPH):
            base = wid * NPH + ph
            pltpu.sync_copy(srcm_hbm.at[base], src_idx)
            pltpu.sync_copy(dstm_hbm.at[base], dst_idx)
            gstart(0, 0)

            def body(i, carry):
                for b in range(NBUF):
                    j = i * NBUF + b
                    b1 = (b + 1) % NBUF

                    @pl.when(j + 1 < CPP)
                    def _():
                        gstart(j + 1, b1)

                    gwait(j, b)
                    pltpu.sync_copy(rows.at[b], agg.at[dst_idx.at[j]],
                                    add=True)
                return carry

            lax.fori_loop(0, CPP // NBUF, body, 0)
        plsc.subcore_barrier()

        @pl.when(c == 0)
        def _():
            pltpu.sync_copy(agg.at[pl.ds(s * RPT, RPT)],
                            out0.at[pl.ds(s * RPT, RPT)])

        @pl.when(c == 1)
        def _():
            pltpu.sync_copy(agg.at[pl.ds(s * RPT, RPT)],
                            out1.at[pl.ds(s * RPT, RPT)])

    return seg_sum


_SEG_SUM = _seg_sum_sc()


def _mlp0(x, w0, b0, w1, b1):
    br = 1000
    grid = N // br

    def body(x_ref, w0_ref, b0_ref, w1_ref, b1_ref, o_ref):
        t = jnp.dot(x_ref[...], w0_ref[...],
                    preferred_element_type=jnp.float32) + b0_ref[...]
        t = jnp.maximum(t, 0.0)
        o_ref[...] = jnp.dot(t, w1_ref[...],
                             preferred_element_type=jnp.float32) + b1_ref[...]

    return pl.pallas_call(
        body,
        grid=(grid,),
        in_specs=[
            pl.BlockSpec((br, FIN), lambda i: (i, 0)),
            pl.BlockSpec((FIN, FM), lambda i: (0, 0)),
            pl.BlockSpec((1, FM), lambda i: (0, 0)),
            pl.BlockSpec((FM, H), lambda i: (0, 0)),
            pl.BlockSpec((1, H), lambda i: (0, 0)),
        ],
        out_specs=pl.BlockSpec((br, H), lambda i: (i, 0)),
        out_shape=jax.ShapeDtypeStruct((N, H), jnp.float32),
    )(x, w0, b0.reshape(1, -1), w1, b1.reshape(1, -1))


def _conv_mlp(h, p0, p1, epsv, w1, b1, w2, b2, scale, shift):
    br = 1000
    grid = N // br

    def body(h_ref, p0_ref, p1_ref, e_ref, w1_ref, b1_ref, w2_ref, b2_ref,
             sc_ref, sh_ref, o_ref):
        hin = h_ref[...] * e_ref[0, 0] + p0_ref[...] + p1_ref[...]
        t = jnp.dot(hin, w1_ref[...],
                    preferred_element_type=jnp.float32) + b1_ref[...]
        t = jnp.maximum(t, 0.0)
        t = jnp.dot(t, w2_ref[...],
                    preferred_element_type=jnp.float32) + b2_ref[...]
        t = jnp.maximum(t, 0.0)
        o_ref[...] = t * sc_ref[...] + sh_ref[...]

    return pl.pallas_call(
        body,
        grid=(grid,),
        in_specs=[
            pl.BlockSpec((br, H), lambda i: (i, 0)),
            pl.BlockSpec((br, H), lambda i: (i, 0)),
            pl.BlockSpec((br, H), lambda i: (i, 0)),
            pl.BlockSpec((1, 1), lambda i: (0, 0)),
            pl.BlockSpec((H, H), lambda i: (0, 0)),
            pl.BlockSpec((1, H), lambda i: (0, 0)),
            pl.BlockSpec((H, H), lambda i: (0, 0)),
            pl.BlockSpec((1, H), lambda i: (0, 0)),
            pl.BlockSpec((1, H), lambda i: (0, 0)),
            pl.BlockSpec((1, H), lambda i: (0, 0)),
        ],
        out_specs=pl.BlockSpec((br, H), lambda i: (i, 0)),
        out_shape=jax.ShapeDtypeStruct((N, H), jnp.float32),
    )(h, p0, p1, epsv, w1, b1.reshape(1, -1), w2, b2.reshape(1, -1),
      scale, shift)


def _conv_mlp_head(h, p0, p1, epsv, w1, b1, w2, b2, scale, shift,
                   batch2d, l1w, l1b, l2w, l2b):
    br = 1000
    grid = N // br

    def body(h_ref, p0_ref, p1_ref, e_ref, w1_ref, b1_ref, w2_ref, b2_ref,
             sc_ref, sh_ref, b_ref, l1w_ref, l1b_ref, l2w_ref, l2b_ref,
             o_ref, o2_ref, sums, cnts):
        i = pl.program_id(0)

        @pl.when(i == 0)
        def _():
            sums[...] = jnp.zeros_like(sums)
            cnts[...] = jnp.zeros_like(cnts)

        hin = h_ref[...] * e_ref[0, 0] + p0_ref[...] + p1_ref[...]
        t = jnp.dot(hin, w1_ref[...],
                    preferred_element_type=jnp.float32) + b1_ref[...]
        t = jnp.maximum(t, 0.0)
        t = jnp.dot(t, w2_ref[...],
                    preferred_element_type=jnp.float32) + b2_ref[...]
        t = jnp.maximum(t, 0.0)
        t = t * sc_ref[...] + sh_ref[...]
        o_ref[...] = t

        onehot = (b_ref[...] == lax.broadcasted_iota(
            jnp.int32, (1, G), 1)).astype(jnp.float32)
        sums[...] += lax.dot_general(
            onehot, t, (((0,), (0,)), ((), ())),
            preferred_element_type=jnp.float32)
        ones = jnp.ones((br, 1), jnp.float32)
        cnts[...] += lax.dot_general(
            onehot, ones, (((0,), (0,)), ((), ())),
            preferred_element_type=jnp.float32)

        @pl.when(i == grid - 1)
        def _():
            pooled = sums[...] / jnp.maximum(cnts[...], 1.0)
            q = jnp.dot(pooled, l1w_ref[...],
                        preferred_element_type=jnp.float32) + l1b_ref[...]
            q = jnp.maximum(q, 0.0)
            logits = jnp.dot(q, l2w_ref[...],
                             preferred_element_type=jnp.float32) + l2b_ref[...]
            m = jnp.max(logits, axis=-1, keepdims=True)
            ex = jnp.exp(logits - m)
            lse = jnp.log(jnp.sum(ex, axis=-1, keepdims=True)) + m
            o2_ref[...] = logits - lse

    return pl.pallas_call(
        body,
        grid=(grid,),
        in_specs=[
            pl.BlockSpec((br, H), lambda i: (i, 0)),
            pl.BlockSpec((br, H), lambda i: (i, 0)),
            pl.BlockSpec((br, H), lambda i: (i, 0)),
            pl.BlockSpec((1, 1), lambda i: (0, 0)),
            pl.BlockSpec((H, H), lambda i: (0, 0)),
            pl.BlockSpec((1, H), lambda i: (0, 0)),
            pl.BlockSpec((H, H), lambda i: (0, 0)),
            pl.BlockSpec((1, H), lambda i: (0, 0)),
            pl.BlockSpec((1, H), lambda i: (0, 0)),
            pl.BlockSpec((1, H), lambda i: (0, 0)),
            pl.BlockSpec((br, 1), lambda i: (i, 0)),
            pl.BlockSpec((H, H), lambda i: (0, 0)),
            pl.BlockSpec((1, H), lambda i: (0, 0)),
            pl.BlockSpec((H, NCLS), lambda i: (0, 0)),
            pl.BlockSpec((1, NCLS), lambda i: (0, 0)),
        ],
        out_specs=(
            pl.BlockSpec((br, H), lambda i: (i, 0)),
            pl.BlockSpec((G, NCLS), lambda i: (0, 0)),
        ),
        out_shape=(
            jax.ShapeDtypeStruct((N, H), jnp.float32),
            jax.ShapeDtypeStruct((G, NCLS), jnp.float32),
        ),
        scratch_shapes=[
            pltpu.VMEM((G, H), jnp.float32),
            pltpu.VMEM((G, 1), jnp.float32),
        ],
    )(h, p0, p1, epsv, w1, b1.reshape(1, -1), w2, b2.reshape(1, -1),
      scale, shift, batch2d, l1w, l1b.reshape(1, -1), l2w, l2b.reshape(1, -1))


def kernel(x, edge_index, batch, params):
    p = params
    src = edge_index[0]
    dst = edge_index[1]

    npad = E_PAD - E
    ar = jnp.arange(npad, dtype=jnp.int32)
    pad_src = (ar * 97) % N
    pad_dst = N + ar % (N_PAD - N)
    srcm = jnp.concatenate([src, pad_src]).reshape(NW * NPH, CPP, CHUNK)
    dstm = jnp.concatenate([dst, pad_dst]).reshape(NW * NPH, CPP, CHUNK)
    zeros = jnp.zeros((N_PAD, H), jnp.float32)

    h = _mlp0(x, p["fn_w0"], p["fn_b0"], p["fn_w1"], p["fn_b1"])

    for i in range(3):
        p0, p1 = _SEG_SUM(h, srcm, dstm, zeros)
        epsv = (1.0 + p[f"eps{i}"]).reshape(1, 1)
        scale = (p[f"c{i}_g"] / jnp.sqrt(p[f"c{i}_rv"] + 1e-5)).reshape(1, -1)
        shift = p[f"c{i}_be"].reshape(1, -1) - p[f"c{i}_rm"].reshape(1, -1) * scale
        if i < 2:
            h = _conv_mlp(h, p0, p1, epsv,
                          p[f"c{i}_w1"], p[f"c{i}_b1"],
                          p[f"c{i}_w2"], p[f"c{i}_b2"], scale, shift)
        else:
            h, logits = _conv_mlp_head(
                h, p0, p1, epsv,
                p[f"c{i}_w1"], p[f"c{i}_b1"],
                p[f"c{i}_w2"], p[f"c{i}_b2"], scale, shift,
                batch.reshape(-1, 1),
                p["lin1_w"], p["lin1_b"], p["lin2_w"], p["lin2_b"])

    return h, logits


# speedup vs baseline: 1.0771x; 1.0771x over previous
import functools

import jax
import jax.numpy as jnp
from jax import lax
from jax.experimental import pallas as pl
from jax.experimental.pallas import tpu as pltpu
from jax.experimental.pallas import tpu_sc as plsc

N = 10000
E = 320000
FIN = 512
FM = 256
H = 128
G = 64
NCLS = 10

NSUB = 16
NCORE = 2
NW = NSUB * NCORE

CHUNK = 128
NPH = 2
NCH = 80
CPP = NCH // NPH
NBUF = 2
E_PAD = NW * NCH * CHUNK
N_PAD = 10112
RPT = N_PAD // NSUB


def _seg_sum_sc():
    mesh = plsc.VectorSubcoreMesh(core_axis_name="c", subcore_axis_name="s")

    @functools.partial(
        pl.kernel,
        out_type=(
            jax.ShapeDtypeStruct((N_PAD, H), jnp.float32),
            jax.ShapeDtypeStruct((N_PAD, H), jnp.float32),
        ),
        mesh=mesh,
        scratch_types=[
            pltpu.VMEM_SHARED((N_PAD, H), jnp.float32),
            pltpu.VMEM((CPP, CHUNK), jnp.int32),
            pltpu.VMEM((CPP, CHUNK), jnp.int32),
            pltpu.VMEM((NBUF, CHUNK, H), jnp.float32),
            [pltpu.SemaphoreType.DMA] * NBUF,
        ],
    )
    def seg_sum(h_hbm, srcm_hbm, dstm_hbm, zeros_hbm, out0, out1,
                agg, src_idx, dst_idx, rows, gsems):
        c = lax.axis_index("c")
        s = lax.axis_index("s")
        wid = c * NSUB + s
        pltpu.sync_copy(zeros_hbm.at[pl.ds(s * RPT, RPT)],
                        agg.at[pl.ds(s * RPT, RPT)])
        plsc.subcore_barrier()

        def gstart(j, b):
            pltpu.make_async_copy(
                h_hbm.at[src_idx.at[j]], rows.at[b], gsems[b]).start()

        def gwait(j, b):
            pltpu.make_async_copy(
                h_hbm.at[src_idx.at[j]], rows.at[b], gsems[b]).wait()

        for ph in range(NPH):
            base = wid * NCH + ph * CPP
            pltpu.sync_copy(srcm_hbm.at[pl.ds(base, CPP)], src_idx)
            pltpu.sync_copy(dstm_hbm.at[pl.ds(base, CPP)], dst_idx)
            gstart(0, 0)
            gstart(1, 1)

            def body(i, carry):
                for b in range(NBUF):
                    j = i * NBUF + b
                    gwait(j, b)
                    pltpu.sync_copy(rows.at[b], agg.at[dst_idx.at[j]],
                                    add=True)

                    @pl.when(j + 2 < CPP)
                    def _():
                        gstart(j + 2, b)
                return carry

            lax.fori_loop(0, CPP // NBUF, body, 0)
        plsc.subcore_barrier()

        @pl.when(c == 0)
        def _():
            pltpu.sync_copy(agg.at[pl.ds(s * RPT, RPT)],
                            out0.at[pl.ds(s * RPT, RPT)])

        @pl.when(c == 1)
        def _():
            pltpu.sync_copy(agg.at[pl.ds(s * RPT, RPT)],
                            out1.at[pl.ds(s * RPT, RPT)])

    return seg_sum


_SEG_SUM = _seg_sum_sc()


def _mlp0(x, w0, b0, w1, b1):
    br = 1000
    grid = N // br

    def body(x_ref, w0_ref, b0_ref, w1_ref, b1_ref, o_ref):
        t = jnp.dot(x_ref[...], w0_ref[...],
                    preferred_element_type=jnp.float32) + b0_ref[...]
        t = jnp.maximum(t, 0.0)
        o_ref[...] = jnp.dot(t, w1_ref[...],
                             preferred_element_type=jnp.float32) + b1_ref[...]

    return pl.pallas_call(
        body,
        grid=(grid,),
        in_specs=[
            pl.BlockSpec((br, FIN), lambda i: (i, 0)),
            pl.BlockSpec((FIN, FM), lambda i: (0, 0)),
            pl.BlockSpec((1, FM), lambda i: (0, 0)),
            pl.BlockSpec((FM, H), lambda i: (0, 0)),
            pl.BlockSpec((1, H), lambda i: (0, 0)),
        ],
        out_specs=pl.BlockSpec((br, H), lambda i: (i, 0)),
        out_shape=jax.ShapeDtypeStruct((N, H), jnp.float32),
    )(x, w0, b0.reshape(1, -1), w1, b1.reshape(1, -1))


def _conv_mlp(h, p0, p1, epsv, w1, b1, w2, b2, scale, shift):
    br = 1000
    grid = N // br

    def body(h_ref, p0_ref, p1_ref, e_ref, w1_ref, b1_ref, w2_ref, b2_ref,
             sc_ref, sh_ref, o_ref):
        hin = h_ref[...] * e_ref[0, 0] + p0_ref[...] + p1_ref[...]
        t = jnp.dot(hin, w1_ref[...],
                    preferred_element_type=jnp.float32) + b1_ref[...]
        t = jnp.maximum(t, 0.0)
        t = jnp.dot(t, w2_ref[...],
                    preferred_element_type=jnp.float32) + b2_ref[...]
        t = jnp.maximum(t, 0.0)
        o_ref[...] = t * sc_ref[...] + sh_ref[...]

    return pl.pallas_call(
        body,
        grid=(grid,),
        in_specs=[
            pl.BlockSpec((br, H), lambda i: (i, 0)),
            pl.BlockSpec((br, H), lambda i: (i, 0)),
            pl.BlockSpec((br, H), lambda i: (i, 0)),
            pl.BlockSpec((1, 1), lambda i: (0, 0)),
            pl.BlockSpec((H, H), lambda i: (0, 0)),
            pl.BlockSpec((1, H), lambda i: (0, 0)),
            pl.BlockSpec((H, H), lambda i: (0, 0)),
            pl.BlockSpec((1, H), lambda i: (0, 0)),
            pl.BlockSpec((1, H), lambda i: (0, 0)),
            pl.BlockSpec((1, H), lambda i: (0, 0)),
        ],
        out_specs=pl.BlockSpec((br, H), lambda i: (i, 0)),
        out_shape=jax.ShapeDtypeStruct((N, H), jnp.float32),
    )(h, p0, p1, epsv, w1, b1.reshape(1, -1), w2, b2.reshape(1, -1),
      scale, shift)


def _conv_mlp_head(h, p0, p1, epsv, w1, b1, w2, b2, scale, shift,
                   batch2d, l1w, l1b, l2w, l2b):
    br = 1000
    grid = N // br

    def body(h_ref, p0_ref, p1_ref, e_ref, w1_ref, b1_ref, w2_ref, b2_ref,
             sc_ref, sh_ref, b_ref, l1w_ref, l1b_ref, l2w_ref, l2b_ref,
             o_ref, o2_ref, sums, cnts):
        i = pl.program_id(0)

        @pl.when(i == 0)
        def _():
            sums[...] = jnp.zeros_like(sums)
            cnts[...] = jnp.zeros_like(cnts)

        hin = h_ref[...] * e_ref[0, 0] + p0_ref[...] + p1_ref[...]
        t = jnp.dot(hin, w1_ref[...],
                    preferred_element_type=jnp.float32) + b1_ref[...]
        t = jnp.maximum(t, 0.0)
        t = jnp.dot(t, w2_ref[...],
                    preferred_element_type=jnp.float32) + b2_ref[...]
        t = jnp.maximum(t, 0.0)
        t = t * sc_ref[...] + sh_ref[...]
        o_ref[...] = t

        onehot = (b_ref[...] == lax.broadcasted_iota(
            jnp.int32, (1, G), 1)).astype(jnp.float32)
        sums[...] += lax.dot_general(
            onehot, t, (((0,), (0,)), ((), ())),
            preferred_element_type=jnp.float32)
        ones = jnp.ones((br, 1), jnp.float32)
        cnts[...] += lax.dot_general(
            onehot, ones, (((0,), (0,)), ((), ())),
            preferred_element_type=jnp.float32)

        @pl.when(i == grid - 1)
        def _():
            pooled = sums[...] / jnp.maximum(cnts[...], 1.0)
            q = jnp.dot(pooled, l1w_ref[...],
                        preferred_element_type=jnp.float32) + l1b_ref[...]
            q = jnp.maximum(q, 0.0)
            logits = jnp.dot(q, l2w_ref[...],
                             preferred_element_type=jnp.float32) + l2b_ref[...]
            m = jnp.max(logits, axis=-1, keepdims=True)
            ex = jnp.exp(logits - m)
            lse = jnp.log(jnp.sum(ex, axis=-1, keepdims=True)) + m
            o2_ref[...] = logits - lse

    return pl.pallas_call(
        body,
        grid=(grid,),
        in_specs=[
            pl.BlockSpec((br, H), lambda i: (i, 0)),
            pl.BlockSpec((br, H), lambda i: (i, 0)),
            pl.BlockSpec((br, H), lambda i: (i, 0)),
            pl.BlockSpec((1, 1), lambda i: (0, 0)),
            pl.BlockSpec((H, H), lambda i: (0, 0)),
            pl.BlockSpec((1, H), lambda i: (0, 0)),
            pl.BlockSpec((H, H), lambda i: (0, 0)),
            pl.BlockSpec((1, H), lambda i: (0, 0)),
            pl.BlockSpec((1, H), lambda i: (0, 0)),
            pl.BlockSpec((1, H), lambda i: (0, 0)),
            pl.BlockSpec((br, 1), lambda i: (i, 0)),
            pl.BlockSpec((H, H), lambda i: (0, 0)),
            pl.BlockSpec((1, H), lambda i: (0, 0)),
            pl.BlockSpec((H, NCLS), lambda i: (0, 0)),
            pl.BlockSpec((1, NCLS), lambda i: (0, 0)),
        ],
        out_specs=(
            pl.BlockSpec((br, H), lambda i: (i, 0)),
            pl.BlockSpec((G, NCLS), lambda i: (0, 0)),
        ),
        out_shape=(
            jax.ShapeDtypeStruct((N, H), jnp.float32),
            jax.ShapeDtypeStruct((G, NCLS), jnp.float32),
        ),
        scratch_shapes=[
            pltpu.VMEM((G, H), jnp.float32),
            pltpu.VMEM((G, 1), jnp.float32),
        ],
    )(h, p0, p1, epsv, w1, b1.reshape(1, -1), w2, b2.reshape(1, -1),
      scale, shift, batch2d, l1w, l1b.reshape(1, -1), l2w, l2b.reshape(1, -1))


def kernel(x, edge_index, batch, params):
    p = params
    src = edge_index[0]
    dst = edge_index[1]

    npad = E_PAD - E
    ar = jnp.arange(npad, dtype=jnp.int32)
    pad_src = (ar * 97) % N
    pad_dst = N + ar % (N_PAD - N)
    srcm = jnp.concatenate([src, pad_src]).reshape(-1, CHUNK)
    dstm = jnp.concatenate([dst, pad_dst]).reshape(-1, CHUNK)
    zeros = jnp.zeros((N_PAD, H), jnp.float32)

    h = _mlp0(x, p["fn_w0"], p["fn_b0"], p["fn_w1"], p["fn_b1"])

    for i in range(3):
        p0, p1 = _SEG_SUM(h, srcm, dstm, zeros)
        epsv = (1.0 + p[f"eps{i}"]).reshape(1, 1)
        scale = (p[f"c{i}_g"] / jnp.sqrt(p[f"c{i}_rv"] + 1e-5)).reshape(1, -1)
        shift = p[f"c{i}_be"].reshape(1, -1) - p[f"c{i}_rm"].reshape(1, -1) * scale
        if i < 2:
            h = _conv_mlp(h, p0, p1, epsv,
                          p[f"c{i}_w1"], p[f"c{i}_b1"],
                          p[f"c{i}_w2"], p[f"c{i}_b2"], scale, shift)
        else:
            h, logits = _conv_mlp_head(
                h, p0, p1, epsv,
                p[f"c{i}_w1"], p[f"c{i}_b1"],
                p[f"c{i}_w2"], p[f"c{i}_b2"], scale, shift,
                batch.reshape(-1, 1),
                p["lin1_w"], p["lin1_b"], p["lin2_w"], p["lin2_b"])

    return h, logits


# speedup vs baseline: 1.1116x; 1.0320x over previous
import functools

import jax
import jax.numpy as jnp
from jax import lax
from jax.experimental import pallas as pl
from jax.experimental.pallas import tpu as pltpu
from jax.experimental.pallas import tpu_sc as plsc

N = 10000
E = 320000
FIN = 512
FM = 256
H = 128
G = 64
NCLS = 10

NSUB = 16
NCORE = 2
NW = NSUB * NCORE

CHUNK = 128
NPH = 2
NCH = 80
CPP = NCH // NPH
NBUF = 2
E_PAD = NW * NCH * CHUNK
N_PAD = 10112
RPT = N_PAD // NSUB


def _seg_sum_sc():
    mesh = plsc.VectorSubcoreMesh(core_axis_name="c", subcore_axis_name="s")

    @functools.partial(
        pl.kernel,
        out_type=(
            jax.ShapeDtypeStruct((N_PAD, H), jnp.float32),
            jax.ShapeDtypeStruct((N_PAD, H), jnp.float32),
        ),
        mesh=mesh,
        scratch_types=[
            pltpu.VMEM_SHARED((N_PAD, H), jnp.float32),
            pltpu.VMEM((CPP, CHUNK), jnp.int32),
            pltpu.VMEM((CPP, CHUNK), jnp.int32),
            pltpu.VMEM((NBUF, CHUNK, H), jnp.float32),
            [pltpu.SemaphoreType.DMA] * NBUF,
        ],
    )
    def seg_sum(h_hbm, srcm_hbm, dstm_hbm, zeros_hbm, out0, out1,
                agg, src_idx, dst_idx, rows, gsems):
        c = lax.axis_index("c")
        s = lax.axis_index("s")
        wid = c * NSUB + s
        pltpu.sync_copy(zeros_hbm.at[pl.ds(s * RPT, RPT)],
                        agg.at[pl.ds(s * RPT, RPT)])
        plsc.subcore_barrier()

        def gstart(j, b):
            pltpu.make_async_copy(
                h_hbm.at[src_idx.at[j]], rows.at[b], gsems[b]).start()

        def gwait(j, b):
            pltpu.make_async_copy(
                h_hbm.at[src_idx.at[j]], rows.at[b], gsems[b]).wait()

        for ph in range(NPH):
            base = wid * NCH + ph * CPP
            pltpu.sync_copy(srcm_hbm.at[pl.ds(base, CPP)], src_idx)
            pltpu.sync_copy(dstm_hbm.at[pl.ds(base, CPP)], dst_idx)
            gstart(0, 0)
            gstart(1, 1)

            def body(i, carry):
                for b in range(NBUF):
                    j = i * NBUF + b
                    gwait(j, b)
                    pltpu.sync_copy(rows.at[b], agg.at[dst_idx.at[j]],
                                    add=True)

                    @pl.when(j + 2 < CPP)
                    def _():
                        gstart(j + 2, b)
                return carry

            lax.fori_loop(0, CPP // NBUF, body, 0)
        plsc.subcore_barrier()

        @pl.when(c == 0)
        def _():
            pltpu.sync_copy(agg.at[pl.ds(s * RPT, RPT)],
                            out0.at[pl.ds(s * RPT, RPT)])

        @pl.when(c == 1)
        def _():
            pltpu.sync_copy(agg.at[pl.ds(s * RPT, RPT)],
                            out1.at[pl.ds(s * RPT, RPT)])

    return seg_sum


_SEG_SUM = _seg_sum_sc()


def _mlp0(x, w0, b0, w1, b1):
    br = 2000
    grid = N // br

    def body(x_ref, w0_ref, b0_ref, w1_ref, b1_ref, o_ref):
        t = jnp.dot(x_ref[...], w0_ref[...],
                    preferred_element_type=jnp.float32) + b0_ref[...]
        t = jnp.maximum(t, 0.0)
        o_ref[...] = jnp.dot(t, w1_ref[...],
                             preferred_element_type=jnp.float32) + b1_ref[...]

    return pl.pallas_call(
        body,
        grid=(grid,),
        in_specs=[
            pl.BlockSpec((br, FIN), lambda i: (i, 0)),
            pl.BlockSpec((FIN, FM), lambda i: (0, 0)),
            pl.BlockSpec((1, FM), lambda i: (0, 0)),
            pl.BlockSpec((FM, H), lambda i: (0, 0)),
            pl.BlockSpec((1, H), lambda i: (0, 0)),
        ],
        out_specs=pl.BlockSpec((br, H), lambda i: (i, 0)),
        out_shape=jax.ShapeDtypeStruct((N, H), jnp.float32),
    )(x, w0, b0.reshape(1, -1), w1, b1.reshape(1, -1))


def _conv_mlp(h, p0, p1, epsv, w1, b1, w2, b2, scale, shift):
    br = 2000
    grid = N // br

    def body(h_ref, p0_ref, p1_ref, e_ref, w1_ref, b1_ref, w2_ref, b2_ref,
             sc_ref, sh_ref, o_ref):
        hin = h_ref[...] * e_ref[0, 0] + p0_ref[...] + p1_ref[...]
        t = jnp.dot(hin, w1_ref[...],
                    preferred_element_type=jnp.float32) + b1_ref[...]
        t = jnp.maximum(t, 0.0)
        t = jnp.dot(t, w2_ref[...],
                    preferred_element_type=jnp.float32) + b2_ref[...]
        t = jnp.maximum(t, 0.0)
        o_ref[...] = t * sc_ref[...] + sh_ref[...]

    return pl.pallas_call(
        body,
        grid=(grid,),
        in_specs=[
            pl.BlockSpec((br, H), lambda i: (i, 0)),
            pl.BlockSpec((br, H), lambda i: (i, 0)),
            pl.BlockSpec((br, H), lambda i: (i, 0)),
            pl.BlockSpec((1, 1), lambda i: (0, 0)),
            pl.BlockSpec((H, H), lambda i: (0, 0)),
            pl.BlockSpec((1, H), lambda i: (0, 0)),
            pl.BlockSpec((H, H), lambda i: (0, 0)),
            pl.BlockSpec((1, H), lambda i: (0, 0)),
            pl.BlockSpec((1, H), lambda i: (0, 0)),
            pl.BlockSpec((1, H), lambda i: (0, 0)),
        ],
        out_specs=pl.BlockSpec((br, H), lambda i: (i, 0)),
        out_shape=jax.ShapeDtypeStruct((N, H), jnp.float32),
    )(h, p0, p1, epsv, w1, b1.reshape(1, -1), w2, b2.reshape(1, -1),
      scale, shift)


def _conv_mlp_head(h, p0, p1, epsv, w1, b1, w2, b2, scale, shift,
                   batch2d, l1w, l1b, l2w, l2b):
    br = 2000
    grid = N // br

    def body(h_ref, p0_ref, p1_ref, e_ref, w1_ref, b1_ref, w2_ref, b2_ref,
             sc_ref, sh_ref, b_ref, l1w_ref, l1b_ref, l2w_ref, l2b_ref,
             o_ref, o2_ref, sums, cnts):
        i = pl.program_id(0)

        @pl.when(i == 0)
        def _():
            sums[...] = jnp.zeros_like(sums)
            cnts[...] = jnp.zeros_like(cnts)

        hin = h_ref[...] * e_ref[0, 0] + p0_ref[...] + p1_ref[...]
        t = jnp.dot(hin, w1_ref[...],
                    preferred_element_type=jnp.float32) + b1_ref[...]
        t = jnp.maximum(t, 0.0)
        t = jnp.dot(t, w2_ref[...],
                    preferred_element_type=jnp.float32) + b2_ref[...]
        t = jnp.maximum(t, 0.0)
        t = t * sc_ref[...] + sh_ref[...]
        o_ref[...] = t

        onehot = (b_ref[...] == lax.broadcasted_iota(
            jnp.int32, (1, G), 1)).astype(jnp.float32)
        sums[...] += lax.dot_general(
            onehot, t, (((0,), (0,)), ((), ())),
            preferred_element_type=jnp.float32)
        ones = jnp.ones((br, 1), jnp.float32)
        cnts[...] += lax.dot_general(
            onehot, ones, (((0,), (0,)), ((), ())),
            preferred_element_type=jnp.float32)

        @pl.when(i == grid - 1)
        def _():
            pooled = sums[...] / jnp.maximum(cnts[...], 1.0)
            q = jnp.dot(pooled, l1w_ref[...],
                        preferred_element_type=jnp.float32) + l1b_ref[...]
            q = jnp.maximum(q, 0.0)
            logits = jnp.dot(q, l2w_ref[...],
                             preferred_element_type=jnp.float32) + l2b_ref[...]
            m = jnp.max(logits, axis=-1, keepdims=True)
            ex = jnp.exp(logits - m)
            lse = jnp.log(jnp.sum(ex, axis=-1, keepdims=True)) + m
            o2_ref[...] = logits - lse

    return pl.pallas_call(
        body,
        grid=(grid,),
        in_specs=[
            pl.BlockSpec((br, H), lambda i: (i, 0)),
            pl.BlockSpec((br, H), lambda i: (i, 0)),
            pl.BlockSpec((br, H), lambda i: (i, 0)),
            pl.BlockSpec((1, 1), lambda i: (0, 0)),
            pl.BlockSpec((H, H), lambda i: (0, 0)),
            pl.BlockSpec((1, H), lambda i: (0, 0)),
            pl.BlockSpec((H, H), lambda i: (0, 0)),
            pl.BlockSpec((1, H), lambda i: (0, 0)),
            pl.BlockSpec((1, H), lambda i: (0, 0)),
            pl.BlockSpec((1, H), lambda i: (0, 0)),
            pl.BlockSpec((br, 1), lambda i: (i, 0)),
            pl.BlockSpec((H, H), lambda i: (0, 0)),
            pl.BlockSpec((1, H), lambda i: (0, 0)),
            pl.BlockSpec((H, NCLS), lambda i: (0, 0)),
            pl.BlockSpec((1, NCLS), lambda i: (0, 0)),
        ],
        out_specs=(
            pl.BlockSpec((br, H), lambda i: (i, 0)),
            pl.BlockSpec((G, NCLS), lambda i: (0, 0)),
        ),
        out_shape=(
            jax.ShapeDtypeStruct((N, H), jnp.float32),
            jax.ShapeDtypeStruct((G, NCLS), jnp.float32),
        ),
        scratch_shapes=[
            pltpu.VMEM((G, H), jnp.float32),
            pltpu.VMEM((G, 1), jnp.float32),
        ],
    )(h, p0, p1, epsv, w1, b1.reshape(1, -1), w2, b2.reshape(1, -1),
      scale, shift, batch2d, l1w, l1b.reshape(1, -1), l2w, l2b.reshape(1, -1))


def kernel(x, edge_index, batch, params):
    p = params
    src = edge_index[0]
    dst = edge_index[1]

    npad = E_PAD - E
    ar = jnp.arange(npad, dtype=jnp.int32)
    pad_src = (ar * 97) % N
    pad_dst = N + ar % (N_PAD - N)
    srcm = jnp.concatenate([src, pad_src]).reshape(-1, CHUNK)
    dstm = jnp.concatenate([dst, pad_dst]).reshape(-1, CHUNK)
    zeros = jnp.zeros((N_PAD, H), jnp.float32)

    h = _mlp0(x, p["fn_w0"], p["fn_b0"], p["fn_w1"], p["fn_b1"])

    for i in range(3):
        p0, p1 = _SEG_SUM(h, srcm, dstm, zeros)
        epsv = (1.0 + p[f"eps{i}"]).reshape(1, 1)
        scale = (p[f"c{i}_g"] / jnp.sqrt(p[f"c{i}_rv"] + 1e-5)).reshape(1, -1)
        shift = p[f"c{i}_be"].reshape(1, -1) - p[f"c{i}_rm"].reshape(1, -1) * scale
        if i < 2:
            h = _conv_mlp(h, p0, p1, epsv,
                          p[f"c{i}_w1"], p[f"c{i}_b1"],
                          p[f"c{i}_w2"], p[f"c{i}_b2"], scale, shift)
        else:
            h, logits = _conv_mlp_head(
                h, p0, p1, epsv,
                p[f"c{i}_w1"], p[f"c{i}_b1"],
                p[f"c{i}_w2"], p[f"c{i}_b2"], scale, shift,
                batch.reshape(-1, 1),
                p["lin1_w"], p["lin1_b"], p["lin2_w"], p["lin2_b"])

    return h, logits


# speedup vs baseline: 1.1290x; 1.0156x over previous
import functools

import jax
import jax.numpy as jnp
from jax import lax
from jax.experimental import pallas as pl
from jax.experimental.pallas import tpu as pltpu
from jax.experimental.pallas import tpu_sc as plsc

N = 10000
E = 320000
FIN = 512
FM = 256
H = 128
G = 64
NCLS = 10

NSUB = 16
NCORE = 2
NW = NSUB * NCORE

CHUNK = 128
NPH = 2
NCH = 80
CPP = NCH // NPH
NBUF = 2
E_PAD = NW * NCH * CHUNK
N_PAD = 10112
RPT = N_PAD // NSUB


def _seg_sum_sc():
    mesh = plsc.VectorSubcoreMesh(core_axis_name="c", subcore_axis_name="s")

    @functools.partial(
        pl.kernel,
        out_type=(
            jax.ShapeDtypeStruct((N_PAD, H), jnp.float32),
            jax.ShapeDtypeStruct((N_PAD, H), jnp.float32),
        ),
        mesh=mesh,
        scratch_types=[
            pltpu.VMEM_SHARED((N_PAD, H), jnp.float32),
            pltpu.VMEM((CPP, CHUNK), jnp.int32),
            pltpu.VMEM((CPP, CHUNK), jnp.int32),
            pltpu.VMEM((NBUF, CHUNK, H), jnp.float32),
            [pltpu.SemaphoreType.DMA] * NBUF,
        ],
    )
    def seg_sum(h_hbm, srcm_hbm, dstm_hbm, zeros_hbm, out0, out1,
                agg, src_idx, dst_idx, rows, gsems):
        c = lax.axis_index("c")
        s = lax.axis_index("s")
        wid = c * NSUB + s
        pltpu.sync_copy(zeros_hbm.at[pl.ds(s * RPT, RPT)],
                        agg.at[pl.ds(s * RPT, RPT)])
        plsc.subcore_barrier()

        def gstart(j, b):
            pltpu.make_async_copy(
                h_hbm.at[src_idx.at[j]], rows.at[b], gsems[b]).start()

        def gwait(j, b):
            pltpu.make_async_copy(
                h_hbm.at[src_idx.at[j]], rows.at[b], gsems[b]).wait()

        for ph in range(NPH):
            base = wid * NCH + ph * CPP
            pltpu.sync_copy(srcm_hbm.at[pl.ds(base, CPP)], src_idx)
            pltpu.sync_copy(dstm_hbm.at[pl.ds(base, CPP)], dst_idx)
            gstart(0, 0)
            gstart(1, 1)

            def body(i, carry):
                for b in range(NBUF):
                    j = i * NBUF + b
                    gwait(j, b)
                    pltpu.sync_copy(rows.at[b], agg.at[dst_idx.at[j]],
                                    add=True)

                    @pl.when(j + 2 < CPP)
                    def _():
                        gstart(j + 2, b)
                return carry

            lax.fori_loop(0, CPP // NBUF, body, 0)
        plsc.subcore_barrier()

        @pl.when(c == 0)
        def _():
            pltpu.sync_copy(agg.at[pl.ds(s * RPT, RPT)],
                            out0.at[pl.ds(s * RPT, RPT)])

        @pl.when(c == 1)
        def _():
            pltpu.sync_copy(agg.at[pl.ds(s * RPT, RPT)],
                            out1.at[pl.ds(s * RPT, RPT)])

    return seg_sum


_SEG_SUM = _seg_sum_sc()


def _mlp0(x, w0, b0, w1, b1):
    br = 5000
    grid = N // br

    def body(x_ref, w0_ref, b0_ref, w1_ref, b1_ref, o_ref):
        t = jnp.dot(x_ref[...], w0_ref[...],
                    preferred_element_type=jnp.float32) + b0_ref[...]
        t = jnp.maximum(t, 0.0)
        o_ref[...] = jnp.dot(t, w1_ref[...],
                             preferred_element_type=jnp.float32) + b1_ref[...]

    return pl.pallas_call(
        body,
        grid=(grid,),
        in_specs=[
            pl.BlockSpec((br, FIN), lambda i: (i, 0)),
            pl.BlockSpec((FIN, FM), lambda i: (0, 0)),
            pl.BlockSpec((1, FM), lambda i: (0, 0)),
            pl.BlockSpec((FM, H), lambda i: (0, 0)),
            pl.BlockSpec((1, H), lambda i: (0, 0)),
        ],
        out_specs=pl.BlockSpec((br, H), lambda i: (i, 0)),
        out_shape=jax.ShapeDtypeStruct((N, H), jnp.float32),
    )(x, w0, b0.reshape(1, -1), w1, b1.reshape(1, -1))


def _conv_mlp(h, p0, p1, epsv, w1, b1, w2, b2, scale, shift):
    br = 5000
    grid = N // br

    def body(h_ref, p0_ref, p1_ref, e_ref, w1_ref, b1_ref, w2_ref, b2_ref,
             sc_ref, sh_ref, o_ref):
        hin = h_ref[...] * e_ref[0, 0] + p0_ref[...] + p1_ref[...]
        t = jnp.dot(hin, w1_ref[...],
                    preferred_element_type=jnp.float32) + b1_ref[...]
        t = jnp.maximum(t, 0.0)
        t = jnp.dot(t, w2_ref[...],
                    preferred_element_type=jnp.float32) + b2_ref[...]
        t = jnp.maximum(t, 0.0)
        o_ref[...] = t * sc_ref[...] + sh_ref[...]

    return pl.pallas_call(
        body,
        grid=(grid,),
        in_specs=[
            pl.BlockSpec((br, H), lambda i: (i, 0)),
            pl.BlockSpec((br, H), lambda i: (i, 0)),
            pl.BlockSpec((br, H), lambda i: (i, 0)),
            pl.BlockSpec((1, 1), lambda i: (0, 0)),
            pl.BlockSpec((H, H), lambda i: (0, 0)),
            pl.BlockSpec((1, H), lambda i: (0, 0)),
            pl.BlockSpec((H, H), lambda i: (0, 0)),
            pl.BlockSpec((1, H), lambda i: (0, 0)),
            pl.BlockSpec((1, H), lambda i: (0, 0)),
            pl.BlockSpec((1, H), lambda i: (0, 0)),
        ],
        out_specs=pl.BlockSpec((br, H), lambda i: (i, 0)),
        out_shape=jax.ShapeDtypeStruct((N, H), jnp.float32),
    )(h, p0, p1, epsv, w1, b1.reshape(1, -1), w2, b2.reshape(1, -1),
      scale, shift)


def _conv_mlp_head(h, p0, p1, epsv, w1, b1, w2, b2, scale, shift,
                   batch2d, l1w, l1b, l2w, l2b):
    br = 5000
    grid = N // br

    def body(h_ref, p0_ref, p1_ref, e_ref, w1_ref, b1_ref, w2_ref, b2_ref,
             sc_ref, sh_ref, b_ref, l1w_ref, l1b_ref, l2w_ref, l2b_ref,
             o_ref, o2_ref, sums, cnts):
        i = pl.program_id(0)

        @pl.when(i == 0)
        def _():
            sums[...] = jnp.zeros_like(sums)
            cnts[...] = jnp.zeros_like(cnts)

        hin = h_ref[...] * e_ref[0, 0] + p0_ref[...] + p1_ref[...]
        t = jnp.dot(hin, w1_ref[...],
                    preferred_element_type=jnp.float32) + b1_ref[...]
        t = jnp.maximum(t, 0.0)
        t = jnp.dot(t, w2_ref[...],
                    preferred_element_type=jnp.float32) + b2_ref[...]
        t = jnp.maximum(t, 0.0)
        t = t * sc_ref[...] + sh_ref[...]
        o_ref[...] = t

        onehot = (b_ref[...] == lax.broadcasted_iota(
            jnp.int32, (1, G), 1)).astype(jnp.float32)
        sums[...] += lax.dot_general(
            onehot, t, (((0,), (0,)), ((), ())),
            preferred_element_type=jnp.float32)
        ones = jnp.ones((br, 1), jnp.float32)
        cnts[...] += lax.dot_general(
            onehot, ones, (((0,), (0,)), ((), ())),
            preferred_element_type=jnp.float32)

        @pl.when(i == grid - 1)
        def _():
            pooled = sums[...] / jnp.maximum(cnts[...], 1.0)
            q = jnp.dot(pooled, l1w_ref[...],
                        preferred_element_type=jnp.float32) + l1b_ref[...]
            q = jnp.maximum(q, 0.0)
            logits = jnp.dot(q, l2w_ref[...],
                             preferred_element_type=jnp.float32) + l2b_ref[...]
            m = jnp.max(logits, axis=-1, keepdims=True)
            ex = jnp.exp(logits - m)
            lse = jnp.log(jnp.sum(ex, axis=-1, keepdims=True)) + m
            o2_ref[...] = logits - lse

    return pl.pallas_call(
        body,
        grid=(grid,),
        in_specs=[
            pl.BlockSpec((br, H), lambda i: (i, 0)),
            pl.BlockSpec((br, H), lambda i: (i, 0)),
            pl.BlockSpec((br, H), lambda i: (i, 0)),
            pl.BlockSpec((1, 1), lambda i: (0, 0)),
            pl.BlockSpec((H, H), lambda i: (0, 0)),
            pl.BlockSpec((1, H), lambda i: (0, 0)),
            pl.BlockSpec((H, H), lambda i: (0, 0)),
            pl.BlockSpec((1, H), lambda i: (0, 0)),
            pl.BlockSpec((1, H), lambda i: (0, 0)),
            pl.BlockSpec((1, H), lambda i: (0, 0)),
            pl.BlockSpec((br, 1), lambda i: (i, 0)),
            pl.BlockSpec((H, H), lambda i: (0, 0)),
            pl.BlockSpec((1, H), lambda i: (0, 0)),
            pl.BlockSpec((H, NCLS), lambda i: (0, 0)),
            pl.BlockSpec((1, NCLS), lambda i: (0, 0)),
        ],
        out_specs=(
            pl.BlockSpec((br, H), lambda i: (i, 0)),
            pl.BlockSpec((G, NCLS), lambda i: (0, 0)),
        ),
        out_shape=(
            jax.ShapeDtypeStruct((N, H), jnp.float32),
            jax.ShapeDtypeStruct((G, NCLS), jnp.float32),
        ),
        scratch_shapes=[
            pltpu.VMEM((G, H), jnp.float32),
            pltpu.VMEM((G, 1), jnp.float32),
        ],
    )(h, p0, p1, epsv, w1, b1.reshape(1, -1), w2, b2.reshape(1, -1),
      scale, shift, batch2d, l1w, l1b.reshape(1, -1), l2w, l2b.reshape(1, -1))


def kernel(x, edge_index, batch, params):
    p = params
    src = edge_index[0]
    dst = edge_index[1]

    npad = E_PAD - E
    ar = jnp.arange(npad, dtype=jnp.int32)
    pad_src = (ar * 97) % N
    pad_dst = N + ar % (N_PAD - N)
    srcm = jnp.concatenate([src, pad_src]).reshape(-1, CHUNK)
    dstm = jnp.concatenate([dst, pad_dst]).reshape(-1, CHUNK)
    zeros = jnp.zeros((N_PAD, H), jnp.float32)

    h = _mlp0(x, p["fn_w0"], p["fn_b0"], p["fn_w1"], p["fn_b1"])

    for i in range(3):
        p0, p1 = _SEG_SUM(h, srcm, dstm, zeros)
        epsv = (1.0 + p[f"eps{i}"]).reshape(1, 1)
        scale = (p[f"c{i}_g"] / jnp.sqrt(p[f"c{i}_rv"] + 1e-5)).reshape(1, -1)
        shift = p[f"c{i}_be"].reshape(1, -1) - p[f"c{i}_rm"].reshape(1, -1) * scale
        if i < 2:
            h = _conv_mlp(h, p0, p1, epsv,
                          p[f"c{i}_w1"], p[f"c{i}_b1"],
                          p[f"c{i}_w2"], p[f"c{i}_b2"], scale, shift)
        else:
            h, logits = _conv_mlp_head(
                h, p0, p1, epsv,
                p[f"c{i}_w1"], p[f"c{i}_b1"],
                p[f"c{i}_w2"], p[f"c{i}_b2"], scale, shift,
                batch.reshape(-1, 1),
                p["lin1_w"], p["lin1_b"], p["lin2_w"], p["lin2_b"])

    return h, logits
